# baseline (speedup 1.0000x reference)
import functools

import jax
import jax.numpy as jnp
from jax import lax
from jax.experimental import pallas as pl
from jax.experimental.pallas import tpu as pltpu
from jax.experimental.pallas import tpu_sc as plsc

V = 32000
NROWS = 2048
P = 16384
NW = 32
KPW = 64
KPAD = 66
MREP_ROWS = 2056
LANES = 16


def _sc_body(probs_hbm, packed_hbm, mrep_hbm, out_hbm,
             r0, r1, r2, acc, pk, mb, s0, s1, s2):
    c = lax.axis_index("c")
    s = lax.axis_index("s")
    w = s * 2 + c
    k0 = 1 + KPW * w

    def row_of(i):
        return jnp.minimum(k0 - 1 + i, NROWS - 1)

    bufs = (r0, r1, r2)
    sems = (s0, s1, s2)

    pltpu.async_copy(probs_hbm.at[row_of(0)], r0, s0)
    pltpu.async_copy(probs_hbm.at[row_of(1)], r1, s1)
    pltpu.async_copy(packed_hbm, pk, s2)
    pltpu.sync_copy(mrep_hbm.at[pl.ds(k0 * LANES, KPAD * LANES)], mb)

    zeros = jnp.zeros((LANES,), jnp.float32)

    @plsc.parallel_loop(0, P // LANES, unroll=8)
    def _zero(j):
        acc[pl.ds(pl.multiple_of(j * LANES, LANES), LANES)] = zeros

    pltpu.make_async_copy(packed_hbm, pk, s2).wait()
    pltpu.make_async_copy(probs_hbm.at[row_of(0)], r0, s0).wait()

    def compute(first, second, i):
        mvec = mb[pl.ds(pl.multiple_of(i * LANES, LANES), LANES)]
        msum = jnp.sum(mvec)

        @pl.when(msum > 0.0)
        def _():
            @plsc.parallel_loop(0, P // LANES, unroll=8)
            def _inner(j):
                off = pl.ds(pl.multiple_of(j * LANES, LANES), LANES)
                pw = pk[off]
                ia = lax.bitwise_and(pw, 0xFFFF)
                ib = lax.shift_right_logical(pw, 16)
                va = plsc.load_gather(first, [ia])
                vb = plsc.load_gather(second, [ib])
                plsc.addupdate(acc.at[off], va * vb)

    @pl.loop(0, KPAD // 3)
    def _outer(g):
        for ph in range(3):
            i = g * 3 + ph
            cur = ph
            nxt = (ph + 1) % 3
            pf = (ph + 2) % 3
            pltpu.async_copy(probs_hbm.at[row_of(i + 2)], bufs[pf], sems[pf])
            pltpu.make_async_copy(
                probs_hbm.at[row_of(i + 1)], bufs[nxt], sems[nxt]).wait()
            compute(bufs[cur], bufs[nxt], i)

    pltpu.make_async_copy(probs_hbm.at[row_of(KPAD + 1)], bufs[(KPAD + 1) % 3],
                          sems[(KPAD + 1) % 3]).wait()

    pltpu.sync_copy(acc, out_hbm.at[w])


def _sc_partials(probs_flat, packed, mrep):
    mesh = plsc.VectorSubcoreMesh(core_axis_name="c", subcore_axis_name="s")
    fn = pl.kernel(
        _sc_body,
        out_type=jax.ShapeDtypeStruct((NW, P), jnp.float32),
        mesh=mesh,
        compiler_params=pltpu.CompilerParams(needs_layout_passes=False),
        scratch_types=[
            pltpu.VMEM((V,), jnp.float32),
            pltpu.VMEM((V,), jnp.float32),
            pltpu.VMEM((V,), jnp.float32),
            pltpu.VMEM((P,), jnp.float32),
            pltpu.VMEM((P,), jnp.int32),
            pltpu.VMEM((KPAD * LANES,), jnp.float32),
            pltpu.SemaphoreType.DMA,
            pltpu.SemaphoreType.DMA,
            pltpu.SemaphoreType.DMA,
        ],
    )
    return fn(probs_flat, packed, mrep)


def _finalize_body(part_ref, mask_ref, tp_ref, toov_ref, out_ref):
    ts = part_ref[0]
    for i in range(1, NW):
        ts = ts + part_ref[i]
    m = mask_ref[...]
    n_pairs = jnp.sum(m[:, :-1] * m[:, 1:])
    n = jnp.maximum(n_pairs, 1.0)
    model_top = jnp.maximum(ts / n, 1e-12)
    top_mass = jnp.sum(model_top)
    model_oov = jnp.clip(1.0 - top_mass, 1e-12, 1.0 - 1e-8)
    tp = jnp.maximum(tp_ref[...], 1e-8)
    kl_top = jnp.sum(model_top * (jnp.log(model_top) - jnp.log(tp)))
    toov = jnp.maximum(toov_ref[0], 1e-8)
    kl_oov = model_oov * (jnp.log(model_oov) - jnp.log(toov))
    out_ref[0] = kl_top + kl_oov


def _finalize(partials, maskf, tp, toov):
    return pl.pallas_call(
        _finalize_body,
        out_shape=jax.ShapeDtypeStruct((1,), jnp.float32),
        in_specs=[
            pl.BlockSpec(memory_space=None),
            pl.BlockSpec(memory_space=None),
            pl.BlockSpec(memory_space=None),
            pl.BlockSpec(memory_space=pltpu.SMEM),
        ],
        out_specs=pl.BlockSpec(memory_space=pltpu.SMEM),
    )(partials, maskf, tp, toov)


@jax.jit
def kernel(probs, mask, pairs, target_probs, target_oov):
    probs_flat = probs.reshape(NROWS, V)
    perm = jnp.argsort(jnp.bitwise_and(pairs[:, 0].astype(jnp.int32), 15),
                       stable=True)
    perm = perm.reshape(LANES, P // LANES).T.reshape(-1)
    pairs_p = pairs[perm]
    tp_p = target_probs[perm]
    a = pairs_p[:, 0].astype(jnp.int32)
    b = pairs_p[:, 1].astype(jnp.int32)
    packed = jnp.bitwise_or(a, jnp.left_shift(b, 16))

    maskf = mask.astype(jnp.float32)
    pm = maskf[:, :-1] * maskf[:, 1:]
    mr = jnp.zeros((MREP_ROWS,), jnp.float32)
    mr = mr.at[1:1024].set(pm[0]).at[1025:2048].set(pm[1])
    mrep = jnp.broadcast_to(mr[:, None], (MREP_ROWS, LANES)).reshape(-1)

    partials = _sc_partials(probs_flat, packed, mrep)

    mask_pad = jnp.pad(maskf, ((0, 6), (0, 0)))
    out = _finalize(
        partials.reshape(NW, 128, 128),
        mask_pad,
        tp_p.reshape(128, 128),
        target_oov.reshape(1),
    )
    return out.reshape(())


# speedup vs baseline: 1.1124x; 1.1124x over previous
import functools

import jax
import jax.numpy as jnp
from jax import lax
from jax.experimental import pallas as pl
from jax.experimental.pallas import tpu as pltpu
from jax.experimental.pallas import tpu_sc as plsc

V = 32000
NROWS = 2048
P = 16384
NW = 32
KPW = 64
KPAD = 66
MREP_ROWS = 2056
LANES = 16


def _sc_body(probs_hbm, packed_hbm, mrep_hbm, out_hbm,
             r0, r1, r2, acc, pk, mb, s0, s1, s2):
    c = lax.axis_index("c")
    s = lax.axis_index("s")
    w = s * 2 + c
    k0 = 1 + KPW * w

    def row_of(i):
        return jnp.minimum(k0 - 1 + i, NROWS - 1)

    bufs = (r0, r1, r2)
    sems = (s0, s1, s2)

    pltpu.async_copy(probs_hbm.at[row_of(0)], r0, s0)
    pltpu.async_copy(probs_hbm.at[row_of(1)], r1, s1)
    pltpu.async_copy(packed_hbm, pk, s2)
    pltpu.sync_copy(mrep_hbm.at[pl.ds(k0 * LANES, KPAD * LANES)], mb)

    zeros = jnp.zeros((LANES,), jnp.float32)

    @plsc.parallel_loop(0, P // LANES, unroll=8)
    def _zero(j):
        acc[pl.ds(pl.multiple_of(j * LANES, LANES), LANES)] = zeros

    pltpu.make_async_copy(packed_hbm, pk, s2).wait()
    pltpu.make_async_copy(probs_hbm.at[row_of(0)], r0, s0).wait()

    def compute(first, second, i):
        mvec = mb[pl.ds(pl.multiple_of(i * LANES, LANES), LANES)]
        msum = jnp.sum(mvec)

        @pl.when(msum > 0.0)
        def _():
            @plsc.parallel_loop(0, P // LANES, unroll=8)
            def _inner(j):
                off = pl.ds(pl.multiple_of(j * LANES, LANES), LANES)
                pw = pk[off]
                ia = lax.bitwise_and(pw, 0xFFFF)
                ib = lax.shift_right_logical(pw, 16)
                va = plsc.load_gather(first, [ia])
                vb = plsc.load_gather(second, [ib])
                plsc.addupdate(acc.at[off], va * vb)

    @pl.loop(0, KPAD // 3)
    def _outer(g):
        for ph in range(3):
            i = g * 3 + ph
            cur = ph
            nxt = (ph + 1) % 3
            pf = (ph + 2) % 3
            pltpu.async_copy(probs_hbm.at[row_of(i + 2)], bufs[pf], sems[pf])
            pltpu.make_async_copy(
                probs_hbm.at[row_of(i + 1)], bufs[nxt], sems[nxt]).wait()
            compute(bufs[cur], bufs[nxt], i)

    pltpu.make_async_copy(probs_hbm.at[row_of(KPAD + 1)], bufs[(KPAD + 1) % 3],
                          sems[(KPAD + 1) % 3]).wait()

    pltpu.sync_copy(acc, out_hbm.at[w])


def _sc_partials(probs_flat, packed, mrep):
    mesh = plsc.VectorSubcoreMesh(core_axis_name="c", subcore_axis_name="s")
    fn = pl.kernel(
        _sc_body,
        out_type=jax.ShapeDtypeStruct((NW, P), jnp.float32),
        mesh=mesh,
        compiler_params=pltpu.CompilerParams(needs_layout_passes=False),
        scratch_types=[
            pltpu.VMEM((V,), jnp.float32),
            pltpu.VMEM((V,), jnp.float32),
            pltpu.VMEM((V,), jnp.float32),
            pltpu.VMEM((P,), jnp.float32),
            pltpu.VMEM((P,), jnp.int32),
            pltpu.VMEM((KPAD * LANES,), jnp.float32),
            pltpu.SemaphoreType.DMA,
            pltpu.SemaphoreType.DMA,
            pltpu.SemaphoreType.DMA,
        ],
    )
    return fn(probs_flat, packed, mrep)


def _finalize_body(part_ref, mask_ref, tp_ref, toov_ref, out_ref):
    ts = part_ref[0]
    for i in range(1, NW):
        ts = ts + part_ref[i]
    m = mask_ref[...]
    n_pairs = jnp.sum(m[:, :-1] * m[:, 1:])
    n = jnp.maximum(n_pairs, 1.0)
    model_top = jnp.maximum(ts / n, 1e-12)
    top_mass = jnp.sum(model_top)
    model_oov = jnp.clip(1.0 - top_mass, 1e-12, 1.0 - 1e-8)
    tp = jnp.maximum(tp_ref[...], 1e-8)
    kl_top = jnp.sum(model_top * (jnp.log(model_top) - jnp.log(tp)))
    toov = jnp.maximum(toov_ref[0], 1e-8)
    kl_oov = model_oov * (jnp.log(model_oov) - jnp.log(toov))
    out_ref[0] = kl_top + kl_oov


def _finalize(partials, maskf, tp, toov):
    return pl.pallas_call(
        _finalize_body,
        out_shape=jax.ShapeDtypeStruct((1,), jnp.float32),
        in_specs=[
            pl.BlockSpec(memory_space=None),
            pl.BlockSpec(memory_space=None),
            pl.BlockSpec(memory_space=None),
            pl.BlockSpec(memory_space=pltpu.SMEM),
        ],
        out_specs=pl.BlockSpec(memory_space=pltpu.SMEM),
    )(partials, maskf, tp, toov)


@jax.jit
def kernel(probs, mask, pairs, target_probs, target_oov):
    probs_flat = probs.reshape(NROWS, V)
    a = pairs[:, 0].astype(jnp.int32)
    b = pairs[:, 1].astype(jnp.int32)
    packed = jnp.bitwise_or(a, jnp.left_shift(b, 16))

    maskf = mask.astype(jnp.float32)
    pm = maskf[:, :-1] * maskf[:, 1:]
    mr = jnp.zeros((MREP_ROWS,), jnp.float32)
    mr = mr.at[1:1024].set(pm[0]).at[1025:2048].set(pm[1])
    mrep = jnp.broadcast_to(mr[:, None], (MREP_ROWS, LANES)).reshape(-1)

    partials = _sc_partials(probs_flat, packed, mrep)

    mask_pad = jnp.pad(maskf, ((0, 6), (0, 0)))
    out = _finalize(
        partials.reshape(NW, 128, 128),
        mask_pad,
        target_probs.reshape(128, 128),
        target_oov.reshape(1),
    )
    return out.reshape(())
